# baseline (speedup 1.0000x reference)
import jax
import jax.numpy as jnp
from jax.experimental import pallas as pl
from jax.experimental.pallas import tpu as pltpu

_K = 1000
_M = 16384
_N = 128
_RC = 200
_NCH = _K // _RC
_NBUF = 2


def _mm_kslab(xt_hbm, w_ref, b_ref, o_hbm, xbuf, acc, insem, outsem):
    def in_copy(c, slot):
        return pltpu.make_async_copy(
            xt_hbm.at[pl.ds(c * _RC, _RC), :], xbuf.at[slot], insem.at[slot]
        )

    for c in range(_NBUF - 1):
        in_copy(c, c % _NBUF).start()
    for c in range(_NCH):
        nxt = c + _NBUF - 1
        if nxt < _NCH:
            in_copy(nxt, nxt % _NBUF).start()
        in_copy(c, c % _NBUF).wait()
        part = jax.lax.dot_general(
            w_ref[pl.ds(c * _RC, _RC), :], xbuf[c % _NBUF],
            (((0,), (0,)), ((), ())),
            preferred_element_type=jnp.float32,
        )
        if c == 0:
            acc[...] = part
        elif c == _NCH - 1:
            acc[...] = jnp.maximum(acc[...] + part + b_ref[...], 0.0)
        else:
            acc[...] = acc[...] + part
    out = pltpu.make_async_copy(acc, o_hbm, outsem)
    out.start()
    out.wait()


@jax.jit
def _run(inputs, weights, bias_col):
    m, k = inputs.shape
    n = weights.shape[1]
    xt = inputs.T
    out_t = pl.pallas_call(
        _mm_kslab,
        in_specs=[
            pl.BlockSpec(memory_space=pltpu.MemorySpace.HBM),
            pl.BlockSpec(memory_space=pltpu.MemorySpace.VMEM),
            pl.BlockSpec(memory_space=pltpu.MemorySpace.VMEM),
        ],
        out_specs=pl.BlockSpec(memory_space=pltpu.MemorySpace.HBM),
        out_shape=jax.ShapeDtypeStruct((n, m), jnp.float32),
        scratch_shapes=[
            pltpu.VMEM((_NBUF, _RC, _M), jnp.float32),
            pltpu.VMEM((n, m), jnp.float32),
            pltpu.SemaphoreType.DMA((_NBUF,)),
            pltpu.SemaphoreType.DMA(()),
        ],
    )(xt, weights, bias_col)
    return out_t.T


def kernel(inputs, kernel, bias):
    return _run(inputs, kernel, bias.reshape(-1, 1))


# speedup vs baseline: 1.9561x; 1.9561x over previous
import jax
import jax.numpy as jnp
from jax.experimental import pallas as pl


def _fused_kernel_t(xt_ref, w_ref, b_ref, o_ref):
    acc = jax.lax.dot_general(
        xt_ref[...], w_ref[...], (((0,), (0,)), ((), ())),
        preferred_element_type=jnp.float32,
    )
    o_ref[...] = jnp.maximum(acc + b_ref[...], 0.0)


@jax.jit
def _run(inputs, weights, bias2d):
    m, k = inputs.shape
    n = weights.shape[1]
    xt = inputs.T
    bn = 2048
    return pl.pallas_call(
        _fused_kernel_t,
        grid=(m // bn,),
        in_specs=[
            pl.BlockSpec((k, bn), lambda i: (0, i)),
            pl.BlockSpec((k, n), lambda i: (0, 0)),
            pl.BlockSpec((1, n), lambda i: (0, 0)),
        ],
        out_specs=pl.BlockSpec((bn, n), lambda i: (i, 0)),
        out_shape=jax.ShapeDtypeStruct((m, n), jnp.float32),
    )(xt, weights, bias2d)


def kernel(inputs, kernel, bias):
    return _run(inputs, kernel, bias.reshape(1, -1))
